# baseline (speedup 1.0000x reference)
import functools

import jax
import jax.numpy as jnp
from jax import lax
from jax.experimental import pallas as pl
from jax.experimental.pallas import tpu as pltpu
from jax.experimental.pallas import tpu_sc as plsc

_NC = 2
_NS = 16


def _sc_body(b_per_w, dummy_hbm, table_hbm, out_hbm, shared, fsem, osem):
    del dummy_hbm
    c = table_hbm.shape[0]
    sid = lax.axis_index("s")
    wid = sid * _NC + lax.axis_index("c")
    base = pl.multiple_of(wid * b_per_w, b_per_w)
    nslab = b_per_w // 8

    nrep = shared.shape[0]

    @pl.when(sid < nrep)
    def _fill():
        for j in range(c):
            pltpu.make_async_copy(
                table_hbm.at[j], shared.at[sid, j], fsem
            ).start()
        for j in range(c):
            pltpu.make_async_copy(
                table_hbm.at[j], shared.at[sid, j], fsem
            ).wait()

    plsc.subcore_barrier()
    rep = lax.rem(sid, nrep)
    for j in range(c):
        for t in range(nslab):
            pltpu.make_async_copy(
                shared.at[rep, j],
                out_hbm.at[j, pl.ds(base + 8 * t, 8), :],
                osem,
            ).start()
    for j in range(c):
        for t in range(nslab):
            pltpu.make_async_copy(
                shared.at[rep, j],
                out_hbm.at[j, pl.ds(base + 8 * t, 8), :],
                osem,
            ).wait()


def kernel(dummy, position_ids):
    b = dummy.shape[0]
    _, c, one, s = position_ids.shape
    table = lax.bitcast_convert_type(position_ids.reshape(c, 1, s), jnp.bfloat16)
    table8 = jnp.broadcast_to(table, (c, 8, s))
    b_per_w = b // (_NC * _NS)
    mesh = plsc.VectorSubcoreMesh(core_axis_name="c", subcore_axis_name="s")
    run = functools.partial(
        pl.kernel,
        mesh=mesh,
        out_type=jax.ShapeDtypeStruct((c, b, s), jnp.bfloat16),
        scratch_types=[
            pltpu.VMEM_SHARED((4, c, 8, s), jnp.bfloat16),
            pltpu.SemaphoreType.DMA,
            pltpu.SemaphoreType.DMA,
        ],
    )(functools.partial(_sc_body, b_per_w))
    out = run(dummy, table8)
    out16 = lax.bitcast_convert_type(out, position_ids.dtype)
    return jnp.transpose(out16, (1, 0, 2)).reshape(b, c, one, s)


# speedup vs baseline: 1.3551x; 1.3551x over previous
import jax
import jax.numpy as jnp
from jax import lax
from jax.experimental import pallas as pl

_BB = 32


def _bcast_kernel(dummy_ref, pos_ref, out_ref):
    del dummy_ref
    c, bb, s = out_ref.shape
    for j in range(c):
        row = pos_ref[pl.ds(j, 1), :]
        out_ref[j] = jnp.broadcast_to(row, (bb, s))


def kernel(dummy, position_ids):
    b = dummy.shape[0]
    _, c, one, s = position_ids.shape
    table = lax.bitcast_convert_type(position_ids.reshape(c, s), jnp.bfloat16)
    idx2d = dummy.reshape(1, b)
    grid = b // _BB
    out = pl.pallas_call(
        _bcast_kernel,
        grid=(grid,),
        in_specs=[
            pl.BlockSpec((1, b), lambda i: (0, 0)),
            pl.BlockSpec((c, s), lambda i: (0, 0)),
        ],
        out_specs=pl.BlockSpec((c, _BB, s), lambda i: (0, i, 0)),
        out_shape=jax.ShapeDtypeStruct((c, b, s), jnp.bfloat16),
    )(idx2d, table)
    out16 = lax.bitcast_convert_type(out, position_ids.dtype)
    return jnp.transpose(out16, (1, 0, 2)).reshape(b, c, one, s)
